# baseline (speedup 1.0000x reference)
import math

import jax
import jax.numpy as jnp
from jax.experimental import pallas as pl
from jax.experimental.pallas import tpu as pltpu

_T = 32768
_D = 768
_E = 8
_EPS = 0.01
_SQRT2 = math.sqrt(2.0)
_BLOCK_T = 8192


def _gating_kernel(x_ref, w_ref, wn_ref, gates_ref, load_out,
                   wc_ref, load_acc):
    i = pl.program_id(0)
    nb = pl.num_programs(0)

    @pl.when(i == 0)
    def _first():
        wc_ref[:_E, :] = w_ref[...]
        wc_ref[_E:, :] = wn_ref[...]
        load_acc[...] = jnp.zeros_like(load_acc)

    xb = x_ref[...]
    wc = wc_ref[...]

    dims = (((1,), (1,)), ((), ()))
    hc = jax.lax.dot_general(xb, wc, dims,
                             preferred_element_type=jnp.float32)
    hct = hc.T
    clean = hct[:_E, :]
    raw = hct[_E:, :]
    noise = jax.nn.softplus(raw) + _EPS
    h = clean + noise

    neg_inf = jnp.float32(-jnp.inf)
    m1 = jnp.max(h, axis=0, keepdims=True)
    eq1 = h == m1
    c1 = jnp.sum(eq1.astype(jnp.float32), axis=0, keepdims=True)
    rest1 = jnp.where(eq1, neg_inf, h)
    r1 = jnp.max(rest1, axis=0, keepdims=True)
    m2 = jnp.where(c1 >= 2.0, m1, r1)
    eq2 = (h == r1) & (~eq1)
    c2 = jnp.sum(eq2.astype(jnp.float32), axis=0, keepdims=True)
    rest2 = jnp.where(eq1 | eq2, neg_inf, h)
    r2 = jnp.max(rest2, axis=0, keepdims=True)
    m3 = jnp.where(
        c1 >= 3.0, m1,
        jnp.where(c1 == 2.0, r1, jnp.where(c2 >= 2.0, r1, r2)))

    keep = h >= m2
    g = jnp.where(keep, jnp.exp(h - m1), 0.0)
    gates = g / jnp.sum(g, axis=0, keepdims=True)
    gates_ref[...] = gates.T

    denom = _SQRT2 * noise + 1e-20
    p_in = 0.5 * (1.0 + jax.lax.erf((clean - m3) / denom))
    p_out = 0.5 * (1.0 + jax.lax.erf((clean - m2) / denom))
    prob = jnp.where(h > m3, p_in, p_out)
    load_acc[...] += jnp.sum(prob, axis=1, keepdims=True)

    @pl.when(i == nb - 1)
    def _emit_load():
        load_out[...] = load_acc[...].T


def kernel(x, W, Wn):
    n_blocks = _T // _BLOCK_T
    gates, load = pl.pallas_call(
        _gating_kernel,
        grid=(n_blocks,),
        in_specs=[
            pl.BlockSpec((_BLOCK_T, _D), lambda i: (i, 0)),
            pl.BlockSpec((_E, _D), lambda i: (0, 0)),
            pl.BlockSpec((_E, _D), lambda i: (0, 0)),
        ],
        out_specs=[
            pl.BlockSpec((_BLOCK_T, _E), lambda i: (i, 0)),
            pl.BlockSpec((1, _E), lambda i: (0, 0)),
        ],
        out_shape=[
            jax.ShapeDtypeStruct((_T, _E), jnp.float32),
            jax.ShapeDtypeStruct((1, _E), jnp.float32),
        ],
        scratch_shapes=[
            pltpu.VMEM((2 * _E, _D), jnp.float32),
            pltpu.VMEM((_E, 1), jnp.float32),
        ],
    )(x, W, Wn)
    return (load.reshape(_E), gates)


# speedup vs baseline: 1.0848x; 1.0848x over previous
import math

import jax
import jax.numpy as jnp
from jax.experimental import pallas as pl
from jax.experimental.pallas import tpu as pltpu

_T = 32768
_D = 768
_E = 8
_EPS = 0.01
_SQRT2 = math.sqrt(2.0)
_BLOCK_T = 4096


def _gating_kernel(x_ref, w_ref, wn_ref, gates_ref, load_out,
                   wc_ref, load_acc):
    i = pl.program_id(0)
    nb = pl.num_programs(0)

    @pl.when(i == 0)
    def _first():
        wc_ref[:_E, :] = w_ref[...]
        wc_ref[_E:, :] = wn_ref[...]
        load_acc[...] = jnp.zeros_like(load_acc)

    xb = x_ref[...]
    wc = wc_ref[...]

    dims = (((1,), (1,)), ((), ()))
    hc = jax.lax.dot_general(xb, wc, dims,
                             preferred_element_type=jnp.float32)
    hct = hc.T
    clean = hct[:_E, :]
    raw = hct[_E:, :]
    noise = jax.nn.softplus(raw) + _EPS
    h = clean + noise

    neg_inf = jnp.float32(-jnp.inf)
    m1 = jnp.max(h, axis=0, keepdims=True)
    eq1 = h == m1
    c1 = jnp.sum(eq1.astype(jnp.float32), axis=0, keepdims=True)
    rest1 = jnp.where(eq1, neg_inf, h)
    r1 = jnp.max(rest1, axis=0, keepdims=True)
    m2 = jnp.where(c1 >= 2.0, m1, r1)
    eq2 = (h == r1) & (~eq1)
    c2 = jnp.sum(eq2.astype(jnp.float32), axis=0, keepdims=True)
    rest2 = jnp.where(eq1 | eq2, neg_inf, h)
    r2 = jnp.max(rest2, axis=0, keepdims=True)
    m3 = jnp.where(
        c1 >= 3.0, m1,
        jnp.where(c1 == 2.0, r1, jnp.where(c2 >= 2.0, r1, r2)))

    keep = h >= m2
    g = jnp.where(keep, jnp.exp(h - m1), 0.0)
    gates = g / jnp.sum(g, axis=0, keepdims=True)
    gates_ref[...] = gates.T

    denom = _SQRT2 * noise + 1e-20
    p_in = 0.5 * (1.0 + jax.lax.erf((clean - m3) / denom))
    p_out = 0.5 * (1.0 + jax.lax.erf((clean - m2) / denom))
    prob = jnp.where(h > m3, p_in, p_out)
    load_acc[...] += jnp.sum(prob, axis=1, keepdims=True)

    @pl.when(i == nb - 1)
    def _emit_load():
        load_out[...] = load_acc[...].T


def kernel(x, W, Wn):
    n_blocks = _T // _BLOCK_T
    gates, load = pl.pallas_call(
        _gating_kernel,
        grid=(n_blocks,),
        in_specs=[
            pl.BlockSpec((_BLOCK_T, _D), lambda i: (i, 0)),
            pl.BlockSpec((_E, _D), lambda i: (0, 0)),
            pl.BlockSpec((_E, _D), lambda i: (0, 0)),
        ],
        out_specs=[
            pl.BlockSpec((_BLOCK_T, _E), lambda i: (i, 0)),
            pl.BlockSpec((1, _E), lambda i: (0, 0)),
        ],
        out_shape=[
            jax.ShapeDtypeStruct((_T, _E), jnp.float32),
            jax.ShapeDtypeStruct((1, _E), jnp.float32),
        ],
        scratch_shapes=[
            pltpu.VMEM((2 * _E, _D), jnp.float32),
            pltpu.VMEM((_E, 1), jnp.float32),
        ],
    )(x, W, Wn)
    return (load.reshape(_E), gates)
